# baseline (speedup 1.0000x reference)
import functools

import jax
import jax.numpy as jnp
from jax.experimental import pallas as pl
from jax.experimental.pallas import tpu as pltpu

H = 128
L = 5
N_TECH = 100000
N_LOC = 50000
N_DEM = 50000
N_FLOW = 100000


def _linear_act_kernel(x_ref, w_ref, b_ref, o_ref, *, act):
    y = jnp.dot(x_ref[...], w_ref[...], preferred_element_type=jnp.float32)
    y = y + b_ref[...]
    if act == "relu":
        y = jnp.maximum(y, 0.0)
    elif act == "tanh":
        y = jnp.tanh(y)
    o_ref[...] = y


def _linear_act(x, W, b, act, block=2048):
    N, din = x.shape
    dout = W.shape[0]
    if din < 8:
        x = jnp.pad(x, ((0, 0), (0, 8 - din)))
        W = jnp.pad(W, ((0, 0), (0, 8 - din)))
        din = 8
    Wt = W.T
    b2 = b.reshape(1, dout)
    grid = (pl.cdiv(N, block),)
    return pl.pallas_call(
        functools.partial(_linear_act_kernel, act=act),
        grid=grid,
        in_specs=[
            pl.BlockSpec((block, din), lambda i: (i, 0)),
            pl.BlockSpec((din, dout), lambda i: (0, 0)),
            pl.BlockSpec((1, dout), lambda i: (0, 0)),
        ],
        out_specs=pl.BlockSpec((block, dout), lambda i: (i, 0)),
        out_shape=jax.ShapeDtypeStruct((N, dout), jnp.float32),
    )(x, Wt, b2)


def _sage_combine_kernel(*refs, K, act):
    s_refs = refs[:K]
    v_refs = refs[K:2 * K]
    x_ref = refs[2 * K]
    w_ref = refs[2 * K + 1]
    b_ref = refs[2 * K + 2]
    o_ref = refs[2 * K + 3]
    acc = jnp.dot(x_ref[...], w_ref[K * H:(K + 1) * H, :],
                  preferred_element_type=jnp.float32)
    for k in range(K):
        s = s_refs[k][...] * v_refs[k][...]
        acc = acc + jnp.dot(s, w_ref[k * H:(k + 1) * H, :],
                            preferred_element_type=jnp.float32)
    acc = acc + b_ref[...]
    if act == "relu":
        acc = jnp.maximum(acc, 0.0)
    else:
        acc = jnp.tanh(acc)
    o_ref[...] = acc


def _sage_combine(ssums, invs, x, Wls, Wr_sum, b_sum, act, block=1024):
    K = len(ssums)
    N = x.shape[0]
    w = jnp.concatenate([Wl.T for Wl in Wls] + [Wr_sum.T], axis=0)
    b2 = b_sum.reshape(1, H)
    grid = (pl.cdiv(N, block),)
    in_specs = (
        [pl.BlockSpec((block, H), lambda i: (i, 0)) for _ in range(K)]
        + [pl.BlockSpec((block, 1), lambda i: (i, 0)) for _ in range(K)]
        + [
            pl.BlockSpec((block, H), lambda i: (i, 0)),
            pl.BlockSpec(((K + 1) * H, H), lambda i: (0, 0)),
            pl.BlockSpec((1, H), lambda i: (0, 0)),
        ]
    )
    return pl.pallas_call(
        functools.partial(_sage_combine_kernel, K=K, act=act),
        grid=grid,
        in_specs=in_specs,
        out_specs=pl.BlockSpec((block, H), lambda i: (i, 0)),
        out_shape=jax.ShapeDtypeStruct((N, H), jnp.float32),
    )(*ssums, *invs, x, w, b2)


def _seg_mean_parts(h_src, ei, n_dst):
    src = ei[0]
    dst = ei[1]
    msg = jnp.take(h_src, src, axis=0)
    ssum = jax.ops.segment_sum(msg, dst, num_segments=n_dst)
    return ssum


def _inv_counts(ei, n_dst):
    dst = ei[1]
    cnt = jax.ops.segment_sum(jnp.ones(dst.shape, jnp.float32), dst,
                              num_segments=n_dst)
    return (1.0 / jnp.maximum(cnt, 1.0)).reshape(n_dst, 1)


def kernel(x_technology, x_location, x_demand, x_flow, ei_powers,
           ei_powered_by, ei_feeds, ei_fed_by, ei_connected_to,
           ei_connected_from, enc_W_technology, enc_b_technology,
           enc_W_location, enc_b_location, enc_W_demand, enc_b_demand,
           enc_W_flow, enc_b_flow, conv_Wl, conv_Wr, conv_b, demand_W,
           demand_b, flow_W, flow_b):
    h_tech = _linear_act(x_technology, enc_W_technology, enc_b_technology, "relu")
    h_loc = _linear_act(x_location, enc_W_location, enc_b_location, "relu")
    h_dem = _linear_act(x_demand, enc_W_demand, enc_b_demand, "relu")
    h_flow = _linear_act(x_flow, enc_W_flow, enc_b_flow, "tanh")

    inv_powers = _inv_counts(ei_powers, N_LOC)
    inv_fed_by = _inv_counts(ei_fed_by, N_LOC)
    inv_conn_to = _inv_counts(ei_connected_to, N_LOC)
    inv_powered_by = _inv_counts(ei_powered_by, N_TECH)
    inv_feeds = _inv_counts(ei_feeds, N_DEM)
    inv_conn_from = _inv_counts(ei_connected_from, N_FLOW)

    for i in range(L):
        s_powers = _seg_mean_parts(h_tech, ei_powers, N_LOC)
        s_fed_by = _seg_mean_parts(h_dem, ei_fed_by, N_LOC)
        s_conn_to = _seg_mean_parts(h_flow, ei_connected_to, N_LOC)
        s_powered_by = _seg_mean_parts(h_loc, ei_powered_by, N_TECH)
        s_feeds = _seg_mean_parts(h_loc, ei_feeds, N_DEM)
        s_conn_from = _seg_mean_parts(h_loc, ei_connected_from, N_FLOW)

        loc_new = _sage_combine(
            [s_powers, s_fed_by, s_conn_to],
            [inv_powers, inv_fed_by, inv_conn_to],
            h_loc,
            [conv_Wl[i, 0], conv_Wl[i, 3], conv_Wl[i, 4]],
            conv_Wr[i, 0] + conv_Wr[i, 3] + conv_Wr[i, 4],
            conv_b[i, 0] + conv_b[i, 3] + conv_b[i, 4],
            "relu")
        tech_new = _sage_combine([s_powered_by], [inv_powered_by], h_tech,
                                 [conv_Wl[i, 1]], conv_Wr[i, 1], conv_b[i, 1],
                                 "relu")
        dem_new = _sage_combine([s_feeds], [inv_feeds], h_dem,
                                [conv_Wl[i, 2]], conv_Wr[i, 2], conv_b[i, 2],
                                "relu")
        flow_new = _sage_combine([s_conn_from], [inv_conn_from], h_flow,
                                 [conv_Wl[i, 5]], conv_Wr[i, 5], conv_b[i, 5],
                                 "tanh")
        h_tech, h_loc, h_dem, h_flow = tech_new, loc_new, dem_new, flow_new

    p_hat = _linear_act(h_tech, demand_W, demand_b, "none")[:, 0]
    flow_pred = _linear_act(h_flow, flow_W, flow_b, "none")[:, 0:1]

    eps = 1e-08
    p_max = x_technology[:, 1] * x_technology[:, 2] * x_technology[:, 3]
    D = jnp.sum(x_demand[:, 0])
    p_bnd = p_max * jax.nn.sigmoid(p_hat)
    S = jnp.sum(p_bnd)
    is_close = jnp.abs(S - D) <= (0.0001 + 1e-05 * jnp.abs(D))
    slack = p_max - p_bnd
    total_slack = jnp.sum(slack) + eps
    shortfall = D - S
    alpha = shortfall / total_slack
    short_out = jnp.where(total_slack < shortfall, p_max, p_bnd + alpha * slack)
    total_bnd = S + eps
    beta = (S - D) / total_bnd
    surp_out = jnp.where(total_bnd < eps, jnp.zeros_like(p_bnd),
                         (1.0 - beta) * p_bnd)
    p_out = jnp.where(is_close, p_bnd, jnp.where(S < D, short_out, surp_out))
    production = p_out[:, None]

    import_cap = x_flow[:, 0:1]
    export_cap = x_flow[:, 1:2]
    flow_out = jnp.where(flow_pred < 0,
                         jax.nn.sigmoid(-flow_pred) * import_cap,
                         jax.nn.sigmoid(flow_pred) * export_cap)
    return (production, flow_out)
